# baseline (speedup 1.0000x reference)
import jax
from jax import lax
import jax.numpy as jnp
from jax.experimental import pallas as pl
from jax.experimental.pallas import tpu as pltpu
from jax.experimental.pallas import tpu_sc as plsc

_WINDOW = 256
_NBUF = 3
_NWORKERS = 32


def kernel(word_ids, table):
    batch, seq = word_ids.shape
    _, embed = table.shape
    nwin = batch // _WINDOW
    steps = (seq * nwin) // _NWORKERS

    idx_t = word_ids.T.astype(jnp.int32)

    mesh = plsc.VectorSubcoreMesh(
        core_axis_name="core", subcore_axis_name="subcore"
    )

    @pl.kernel(
        out_type=jax.ShapeDtypeStruct((seq, batch, embed), table.dtype),
        mesh=mesh,
        scratch_types=[
            pltpu.VMEM((_WINDOW,), jnp.int32),
            pltpu.VMEM((_WINDOW,), jnp.int32),
            pltpu.VMEM((_WINDOW,), jnp.int32),
            pltpu.VMEM((_NBUF, _WINDOW, embed), table.dtype),
            pltpu.SemaphoreType.DMA((_NBUF,)),
            pltpu.SemaphoreType.DMA((_NBUF,)),
            pltpu.SemaphoreType.DMA((_NBUF,)),
        ],
    )
    def _gather(
        tab_hbm,
        idx_hbm,
        out_hbm,
        idx_v0,
        idx_v1,
        idx_v2,
        rows_v,
        sem_i,
        sem_g,
        sem_o,
    ):
        idx_bufs = (idx_v0, idx_v1, idx_v2)
        wid = lax.axis_index("subcore") * 2 + lax.axis_index("core")

        def win(t):
            w = wid + _NWORKERS * t
            return w // nwin, w % nwin

        def idx_copy(t, b):
            s, j = win(t)
            return pltpu.make_async_copy(
                idx_hbm.at[s, pl.ds(j * _WINDOW, _WINDOW)],
                idx_bufs[b],
                sem_i.at[b],
            )

        def gather_copy(b):
            return pltpu.make_async_copy(
                tab_hbm.at[idx_bufs[b]], rows_v.at[b], sem_g.at[b]
            )

        def out_copy(t, b):
            s, j = win(t)
            return pltpu.make_async_copy(
                rows_v.at[b],
                out_hbm.at[s, pl.ds(j * _WINDOW, _WINDOW), :],
                sem_o.at[b],
            )

        for b in range(_NBUF):
            idx_copy(b, b).start()
        idx_copy(0, 0).wait()
        gather_copy(0).start()

        for t in range(steps):
            b = t % _NBUF
            nb = (t + 1) % _NBUF
            gather_copy(b).wait()
            out_copy(t, b).start()
            if t + 1 < steps:
                idx_copy(t + 1, nb).wait()
                if t >= _NBUF - 1:
                    out_copy(t + 1 - _NBUF, nb).wait()
                gather_copy(nb).start()
                if t + _NBUF < steps:
                    idx_copy(t + _NBUF, b).start()
        for t in range(steps - _NBUF, steps):
            out_copy(t, t % _NBUF).wait()

    y = _gather(table, idx_t)
    return jnp.transpose(y, (1, 0, 2))


# speedup vs baseline: 1.0022x; 1.0022x over previous
import jax
from jax import lax
import jax.numpy as jnp
from jax.experimental import pallas as pl
from jax.experimental.pallas import tpu as pltpu
from jax.experimental.pallas import tpu_sc as plsc

_WINDOW = 256
_NWORKERS = 32


def kernel(word_ids, table):
    batch, seq = word_ids.shape
    _, embed = table.shape
    nwin = batch // _WINDOW
    steps = (seq * nwin) // _NWORKERS

    idx_t = word_ids.T.astype(jnp.int32)

    mesh = plsc.VectorSubcoreMesh(
        core_axis_name="core", subcore_axis_name="subcore"
    )

    @pl.kernel(
        out_type=jax.ShapeDtypeStruct((seq, batch, embed), table.dtype),
        mesh=mesh,
        scratch_types=[
            pltpu.VMEM((_WINDOW,), jnp.int32),
            pltpu.VMEM((_WINDOW,), jnp.int32),
            pltpu.VMEM((2, _WINDOW, embed), table.dtype),
            pltpu.SemaphoreType.DMA((2,)),
            pltpu.SemaphoreType.DMA((2,)),
            pltpu.SemaphoreType.DMA((2,)),
        ],
    )
    def _gather(
        tab_hbm, idx_hbm, out_hbm, idx_v0, idx_v1, rows_v, sem_i, sem_g, sem_o
    ):
        idx_bufs = (idx_v0, idx_v1)
        wid = lax.axis_index("subcore") * 2 + lax.axis_index("core")

        def win(t):
            w = wid + _NWORKERS * t
            return w // nwin, w % nwin

        def idx_copy(t, b):
            s, j = win(t)
            return pltpu.make_async_copy(
                idx_hbm.at[s, pl.ds(j * _WINDOW, _WINDOW)],
                idx_bufs[b],
                sem_i.at[b],
            )

        def gather_copy(b):
            return pltpu.make_async_copy(
                tab_hbm.at[idx_bufs[b]], rows_v.at[b], sem_g.at[b]
            )

        def out_copy(t, b):
            s, j = win(t)
            return pltpu.make_async_copy(
                rows_v.at[b],
                out_hbm.at[s, pl.ds(j * _WINDOW, _WINDOW), :],
                sem_o.at[b],
            )

        idx_copy(0, 0).start()
        idx_copy(1, 1).start()
        idx_copy(0, 0).wait()
        gather_copy(0).start()

        for t in range(steps):
            b = t % 2
            nb = (t + 1) % 2
            gather_copy(b).wait()
            out_copy(t, b).start()
            if t + 1 < steps:
                idx_copy(t + 1, nb).wait()
                if t >= 1:
                    out_copy(t - 1, nb).wait()
                gather_copy(nb).start()
                if t + 2 < steps:
                    idx_copy(t + 2, b).start()
        out_copy(steps - 2, (steps - 2) % 2).wait()
        out_copy(steps - 1, (steps - 1) % 2).wait()

    y = _gather(table, idx_t)
    return jnp.transpose(y, (1, 0, 2))
